# baseline (speedup 1.0000x reference)
import functools

import jax
import jax.numpy as jnp
import numpy as np
from jax.experimental import pallas as pl
from jax.experimental.pallas import tpu as pltpu

B, L, D = 1, 2048, 2048
E, K, F = 8, 2, 1024
S = 2 * F
H = 4
HD = D // H


def _dot(a, b):
    return jnp.dot(a, b, preferred_element_type=jnp.float32)


def _dotf(a, b):
    return jnp.dot(a, b, preferred_element_type=jnp.float32,
                   precision=jax.lax.Precision.HIGHEST)


def _sigmoid(x):
    return 1.0 / (1.0 + jnp.exp(-x))


def _qkv_kernel(x_ref, nw_ref, w_ref, b_ref, o_ref):
    x = x_ref[...]
    v = jnp.mean(x * x, axis=-1, keepdims=True)
    xn = nw_ref[...] * (x * jax.lax.rsqrt(v + 1e-6))
    o_ref[...] = _dotf(xn, w_ref[...].T) + b_ref[...]


def _qkv(x, nw, w, bias):
    RB, CB = 8, 6
    rb, cb = L // RB, (3 * D) // CB
    return pl.pallas_call(
        _qkv_kernel,
        grid=(CB, RB),
        in_specs=[
            pl.BlockSpec((rb, D), lambda c, r: (r, 0)),
            pl.BlockSpec((1, D), lambda c, r: (0, 0)),
            pl.BlockSpec((cb, D), lambda c, r: (c, 0)),
            pl.BlockSpec((1, cb), lambda c, r: (0, c)),
        ],
        out_specs=pl.BlockSpec((rb, cb), lambda c, r: (r, c)),
        out_shape=jax.ShapeDtypeStruct((L, 3 * D), jnp.float32),
    )(x, nw.reshape(1, D), w, bias.reshape(1, 3 * D))


def _attn_kernel(q_ref, k_ref, v_ref, o_ref):
    q = q_ref[...]
    k = k_ref[...]
    s = _dotf(q, k.T) * (1.0 / np.sqrt(HD))
    s = s - jnp.max(s, axis=-1, keepdims=True)
    p = jnp.exp(s)
    p = p / jnp.sum(p, axis=-1, keepdims=True)
    o_ref[...] = _dotf(p, v_ref[...])


def _attn(qkv):
    QB = 2
    qb = L // QB
    return pl.pallas_call(
        _attn_kernel,
        grid=(H, QB),
        in_specs=[
            pl.BlockSpec((qb, HD), lambda h, q: (q, h)),
            pl.BlockSpec((L, HD), lambda h, q: (0, H + h)),
            pl.BlockSpec((L, HD), lambda h, q: (0, 2 * H + h)),
        ],
        out_specs=pl.BlockSpec((qb, HD), lambda h, q: (q, h)),
        out_shape=jax.ShapeDtypeStruct((L, D), jnp.float32),
    )(qkv, qkv, qkv)


def _outproj_kernel(ctx_ref, w_ref, b_ref, x_ref, o_ref):
    o_ref[...] = x_ref[...] + _dotf(ctx_ref[...], w_ref[...].T) + b_ref[...]


def _outproj(ctx, w, bias, x):
    RB = 8
    rb = L // RB
    return pl.pallas_call(
        _outproj_kernel,
        grid=(RB,),
        in_specs=[
            pl.BlockSpec((rb, D), lambda r: (r, 0)),
            pl.BlockSpec((D, D), lambda r: (0, 0)),
            pl.BlockSpec((1, D), lambda r: (0, 0)),
            pl.BlockSpec((rb, D), lambda r: (r, 0)),
        ],
        out_specs=pl.BlockSpec((rb, D), lambda r: (r, 0)),
        out_shape=jax.ShapeDtypeStruct((L, D), jnp.float32),
    )(ctx, w, bias.reshape(1, D), x)


def _gate_kernel(hs_ref, gnw_ref, gw_ref, ogw_ref, ogb_ref,
                 base_ref, coef_ref, gl_ref):
    hs = hs_ref[...]
    v = jnp.mean(hs * hs, axis=-1, keepdims=True)
    base = hs * jax.lax.rsqrt(v + 1e-6)
    base_ref[...] = base
    logits = _dotf(base * gnw_ref[...], gw_ref[...].T)
    m = jnp.max(logits, axis=-1, keepdims=True)
    p = jnp.exp(logits - m)
    scores = p / jnp.sum(p, axis=-1, keepdims=True)
    iota = jax.lax.broadcasted_iota(jnp.int32, (L, E), 1)
    m1 = jnp.max(scores, axis=-1, keepdims=True)
    i1 = jnp.min(jnp.where(scores == m1, iota, E), axis=-1, keepdims=True)
    s2 = jnp.where(iota == i1, -jnp.inf, scores)
    m2 = jnp.max(s2, axis=-1, keepdims=True)
    i2 = jnp.min(jnp.where(s2 == m2, iota, E), axis=-1, keepdims=True)
    wsum = m1 + m2 + 1e-20
    coef_ref[...] = (jnp.where(iota == i1, m1, 0.0)
                     + jnp.where(iota == i2, m2, 0.0)) / wsum
    gl = jnp.sum(hs * ogw_ref[...], axis=-1, keepdims=True) + ogb_ref[0, 0]
    gl_ref[...] = jnp.broadcast_to(gl, (L, 128))


def _gate(hs, gnw, gw, ogw, ogb):
    return pl.pallas_call(
        _gate_kernel,
        grid=(1,),
        in_specs=[
            pl.BlockSpec((L, D), lambda i: (0, 0)),
            pl.BlockSpec((1, D), lambda i: (0, 0)),
            pl.BlockSpec((E, D), lambda i: (0, 0)),
            pl.BlockSpec((1, D), lambda i: (0, 0)),
            pl.BlockSpec((1, 1), lambda i: (0, 0)),
        ],
        out_specs=[
            pl.BlockSpec((L, D), lambda i: (0, 0)),
            pl.BlockSpec((L, E), lambda i: (0, 0)),
            pl.BlockSpec((L, 128), lambda i: (0, 0)),
        ],
        out_shape=[
            jax.ShapeDtypeStruct((L, D), jnp.float32),
            jax.ShapeDtypeStruct((L, E), jnp.float32),
            jax.ShapeDtypeStruct((L, 128), jnp.float32),
        ],
    )(hs, gnw.reshape(1, D), gw, ogw.reshape(1, D), ogb.reshape(1, 1))


def _moe_dense_kernel(base_ref, coef_ref, enw_ref, wg_ref, wu_ref, wd_ref, y_ref):
    e = pl.program_id(1)
    xn = (base_ref[...] * enw_ref[0]).astype(jnp.bfloat16)
    g = _dot(xn, wg_ref[0].T)
    g = g * _sigmoid(g)
    u = _dot(xn, wu_ref[0].T)
    eo = _dot((g * u).astype(jnp.bfloat16), wd_ref[0].T)
    iota = jax.lax.broadcasted_iota(jnp.int32, (1, E), 1)
    ce = jnp.sum(coef_ref[...] * (iota == e), axis=-1, keepdims=True)
    contrib = ce * eo

    @pl.when(e == 0)
    def _():
        y_ref[...] = contrib

    @pl.when(e > 0)
    def _():
        y_ref[...] += contrib


def _moe_dense(base, coef, enw, wg, wu, wd):
    RB = 4
    rb = L // RB
    return pl.pallas_call(
        _moe_dense_kernel,
        grid=(RB, E),
        in_specs=[
            pl.BlockSpec((rb, D), lambda r, e: (r, 0)),
            pl.BlockSpec((rb, E), lambda r, e: (r, 0)),
            pl.BlockSpec((1, D), lambda r, e: (0, e)),
            pl.BlockSpec((1, F, D), lambda r, e: (e, 0, 0)),
            pl.BlockSpec((1, F, D), lambda r, e: (e, 0, 0)),
            pl.BlockSpec((1, D, F), lambda r, e: (e, 0, 0)),
        ],
        out_specs=pl.BlockSpec((rb, D), lambda r, e: (r, 0)),
        out_shape=jax.ShapeDtypeStruct((L, D), jnp.float32),
    )(base, coef, enw.reshape(1, E * D),
      wg.astype(jnp.bfloat16), wu.astype(jnp.bfloat16), wd.astype(jnp.bfloat16))


def _shared1_kernel(x_ref, nw_ref, wg_ref, wu_ref, o_ref):
    x = x_ref[...]
    v = jnp.mean(x * x, axis=-1, keepdims=True)
    xn = (nw_ref[...] * (x * jax.lax.rsqrt(v + 1e-6))).astype(jnp.bfloat16)
    g = _dot(xn, wg_ref[...].T)
    o_ref[...] = g * _sigmoid(g) * _dot(xn, wu_ref[...].T)


def _shared1(x, nw, wg, wu):
    RB, CB = 8, 2
    rb, cb = L // RB, S // CB
    return pl.pallas_call(
        _shared1_kernel,
        grid=(CB, RB),
        in_specs=[
            pl.BlockSpec((rb, D), lambda c, r: (r, 0)),
            pl.BlockSpec((1, D), lambda c, r: (0, 0)),
            pl.BlockSpec((cb, D), lambda c, r: (c, 0)),
            pl.BlockSpec((cb, D), lambda c, r: (c, 0)),
        ],
        out_specs=pl.BlockSpec((rb, cb), lambda c, r: (r, c)),
        out_shape=jax.ShapeDtypeStruct((L, S), jnp.float32),
    )(x, nw.reshape(1, D), wg.astype(jnp.bfloat16), wu.astype(jnp.bfloat16))


def _final_kernel(s1_ref, wd_ref, ident_ref, y_ref, gl_ref, o_ref):
    shared = _dot(s1_ref[...].astype(jnp.bfloat16), wd_ref[...].T)
    ident = ident_ref[...]
    gate = _sigmoid(gl_ref[:, 0:1])
    o_ref[...] = ident + gate * (y_ref[...] + shared)


def _final(s1, wd, ident, y, gl):
    RB = 8
    rb = L // RB
    return pl.pallas_call(
        _final_kernel,
        grid=(RB,),
        in_specs=[
            pl.BlockSpec((rb, S), lambda r: (r, 0)),
            pl.BlockSpec((D, S), lambda r: (0, 0)),
            pl.BlockSpec((rb, D), lambda r: (r, 0)),
            pl.BlockSpec((rb, D), lambda r: (r, 0)),
            pl.BlockSpec((rb, 128), lambda r: (r, 0)),
        ],
        out_specs=pl.BlockSpec((rb, D), lambda r: (r, 0)),
        out_shape=jax.ShapeDtypeStruct((L, D), jnp.float32),
    )(s1, wd.astype(jnp.bfloat16), ident, y, gl)


def kernel(hidden_states, context_norm_w, attn_in_w, attn_in_b, attn_out_w,
           attn_out_b, gate_norm_w, gate_w, expert_norm_w, expert_wg,
           expert_wu, expert_wd, shared_norm_w, shared_wg, shared_wu,
           shared_wd, out_gate_w, out_gate_b):
    x = hidden_states.reshape(L, D)
    qkv = _qkv(x, context_norm_w, attn_in_w, attn_in_b)
    ctx = _attn(qkv)
    ident = _outproj(ctx, attn_out_w, attn_out_b, x)
    base, coef, gl = _gate(ident, gate_norm_w, gate_w, out_gate_w, out_gate_b)
    y = _moe_dense(base, coef, expert_norm_w, expert_wg, expert_wu, expert_wd)
    s1 = _shared1(ident, shared_norm_w, shared_wg, shared_wu)
    out = _final(s1, shared_wd, ident, y, gl)
    return out.reshape(B, L, D)


# speedup vs baseline: 1.7839x; 1.7839x over previous
import functools

import jax
import jax.numpy as jnp
import numpy as np
from jax.experimental import pallas as pl
from jax.experimental.pallas import tpu as pltpu

B, L, D = 1, 2048, 2048
E, K, F = 8, 2, 1024
S = 2 * F
H = 4
HD = D // H


def _dot(a, b):
    return jnp.dot(a.astype(jnp.bfloat16), b.astype(jnp.bfloat16),
                   preferred_element_type=jnp.float32)


def _sigmoid(x):
    return 1.0 / (1.0 + jnp.exp(-x))


def _qkv_kernel(x_ref, nw_ref, w_ref, b_ref, o_ref):
    x = x_ref[...]
    v = jnp.mean(x * x, axis=-1, keepdims=True)
    xn = nw_ref[...] * (x * jax.lax.rsqrt(v + 1e-6))
    o_ref[...] = _dot(xn, w_ref[...].T) + b_ref[...]


def _qkv(x, nw, w, bias):
    RB, CB = 8, 6
    rb, cb = L // RB, (3 * D) // CB
    return pl.pallas_call(
        _qkv_kernel,
        grid=(CB, RB),
        in_specs=[
            pl.BlockSpec((rb, D), lambda c, r: (r, 0)),
            pl.BlockSpec((1, D), lambda c, r: (0, 0)),
            pl.BlockSpec((cb, D), lambda c, r: (c, 0)),
            pl.BlockSpec((1, cb), lambda c, r: (0, c)),
        ],
        out_specs=pl.BlockSpec((rb, cb), lambda c, r: (r, c)),
        out_shape=jax.ShapeDtypeStruct((L, 3 * D), jnp.float32),
    )(x, nw.reshape(1, D), w, bias.reshape(1, 3 * D))


def _attn_kernel(q_ref, k_ref, v_ref, o_ref):
    q = q_ref[...]
    k = k_ref[...]
    s = _dot(q, k.T) * (1.0 / np.sqrt(HD))
    s = s - jnp.max(s, axis=-1, keepdims=True)
    p = jnp.exp(s)
    p = p / jnp.sum(p, axis=-1, keepdims=True)
    o_ref[...] = _dot(p, v_ref[...])


def _attn(qkv):
    QB = 2
    qb = L // QB
    return pl.pallas_call(
        _attn_kernel,
        grid=(H, QB),
        in_specs=[
            pl.BlockSpec((qb, HD), lambda h, q: (q, h)),
            pl.BlockSpec((L, HD), lambda h, q: (0, H + h)),
            pl.BlockSpec((L, HD), lambda h, q: (0, 2 * H + h)),
        ],
        out_specs=pl.BlockSpec((qb, HD), lambda h, q: (q, h)),
        out_shape=jax.ShapeDtypeStruct((L, D), jnp.float32),
    )(qkv, qkv, qkv)


def _outproj_kernel(ctx_ref, w_ref, b_ref, x_ref, o_ref):
    o_ref[...] = x_ref[...] + _dot(ctx_ref[...], w_ref[...].T) + b_ref[...]


def _outproj(ctx, w, bias, x):
    RB = 8
    rb = L // RB
    return pl.pallas_call(
        _outproj_kernel,
        grid=(RB,),
        in_specs=[
            pl.BlockSpec((rb, D), lambda r: (r, 0)),
            pl.BlockSpec((D, D), lambda r: (0, 0)),
            pl.BlockSpec((1, D), lambda r: (0, 0)),
            pl.BlockSpec((rb, D), lambda r: (r, 0)),
        ],
        out_specs=pl.BlockSpec((rb, D), lambda r: (r, 0)),
        out_shape=jax.ShapeDtypeStruct((L, D), jnp.float32),
    )(ctx, w, bias.reshape(1, D), x)


def _gate_kernel(hs_ref, gnw_ref, gw_ref, ogw_ref, ogb_ref,
                 base_ref, coef_ref, gl_ref):
    hs = hs_ref[...]
    v = jnp.mean(hs * hs, axis=-1, keepdims=True)
    base = hs * jax.lax.rsqrt(v + 1e-6)
    base_ref[...] = base
    logits = _dot(base * gnw_ref[...], gw_ref[...].T)
    m = jnp.max(logits, axis=-1, keepdims=True)
    p = jnp.exp(logits - m)
    scores = p / jnp.sum(p, axis=-1, keepdims=True)
    iota = jax.lax.broadcasted_iota(jnp.int32, (L, E), 1)
    m1 = jnp.max(scores, axis=-1, keepdims=True)
    i1 = jnp.min(jnp.where(scores == m1, iota, E), axis=-1, keepdims=True)
    s2 = jnp.where(iota == i1, -jnp.inf, scores)
    m2 = jnp.max(s2, axis=-1, keepdims=True)
    i2 = jnp.min(jnp.where(s2 == m2, iota, E), axis=-1, keepdims=True)
    wsum = m1 + m2 + 1e-20
    coef_ref[...] = (jnp.where(iota == i1, m1, 0.0)
                     + jnp.where(iota == i2, m2, 0.0)) / wsum
    gl = jnp.sum(hs * ogw_ref[...], axis=-1, keepdims=True) + ogb_ref[0, 0]
    gl_ref[...] = jnp.broadcast_to(gl, (L, 128))


def _gate(hs, gnw, gw, ogw, ogb):
    return pl.pallas_call(
        _gate_kernel,
        grid=(1,),
        in_specs=[
            pl.BlockSpec((L, D), lambda i: (0, 0)),
            pl.BlockSpec((1, D), lambda i: (0, 0)),
            pl.BlockSpec((E, D), lambda i: (0, 0)),
            pl.BlockSpec((1, D), lambda i: (0, 0)),
            pl.BlockSpec((1, 1), lambda i: (0, 0)),
        ],
        out_specs=[
            pl.BlockSpec((L, D), lambda i: (0, 0)),
            pl.BlockSpec((L, E), lambda i: (0, 0)),
            pl.BlockSpec((L, 128), lambda i: (0, 0)),
        ],
        out_shape=[
            jax.ShapeDtypeStruct((L, D), jnp.float32),
            jax.ShapeDtypeStruct((L, E), jnp.float32),
            jax.ShapeDtypeStruct((L, 128), jnp.float32),
        ],
    )(hs, gnw.reshape(1, D), gw, ogw.reshape(1, D), ogb.reshape(1, 1))


def _moe_dense_kernel(base_ref, coef_ref, enw_ref, wg_ref, wu_ref, wd_ref, y_ref):
    e = pl.program_id(1)
    xn = (base_ref[...] * enw_ref[0]).astype(jnp.bfloat16)
    g = _dot(xn, wg_ref[0].T)
    g = g * _sigmoid(g)
    u = _dot(xn, wu_ref[0].T)
    eo = _dot((g * u).astype(jnp.bfloat16), wd_ref[0].T)
    iota = jax.lax.broadcasted_iota(jnp.int32, (1, E), 1)
    ce = jnp.sum(coef_ref[...] * (iota == e), axis=-1, keepdims=True)
    contrib = ce * eo

    @pl.when(e == 0)
    def _():
        y_ref[...] = contrib

    @pl.when(e > 0)
    def _():
        y_ref[...] += contrib


def _moe_dense(base, coef, enw, wg, wu, wd):
    RB = 4
    rb = L // RB
    return pl.pallas_call(
        _moe_dense_kernel,
        grid=(RB, E),
        in_specs=[
            pl.BlockSpec((rb, D), lambda r, e: (r, 0)),
            pl.BlockSpec((rb, E), lambda r, e: (r, 0)),
            pl.BlockSpec((1, D), lambda r, e: (0, e)),
            pl.BlockSpec((1, F, D), lambda r, e: (e, 0, 0)),
            pl.BlockSpec((1, F, D), lambda r, e: (e, 0, 0)),
            pl.BlockSpec((1, D, F), lambda r, e: (e, 0, 0)),
        ],
        out_specs=pl.BlockSpec((rb, D), lambda r, e: (r, 0)),
        out_shape=jax.ShapeDtypeStruct((L, D), jnp.float32),
    )(base, coef, enw.reshape(1, E * D),
      wg.astype(jnp.bfloat16), wu.astype(jnp.bfloat16), wd.astype(jnp.bfloat16))


def _shared1_kernel(x_ref, nw_ref, wg_ref, wu_ref, o_ref):
    x = x_ref[...]
    v = jnp.mean(x * x, axis=-1, keepdims=True)
    xn = (nw_ref[...] * (x * jax.lax.rsqrt(v + 1e-6))).astype(jnp.bfloat16)
    g = _dot(xn, wg_ref[...].T)
    o_ref[...] = g * _sigmoid(g) * _dot(xn, wu_ref[...].T)


def _shared1(x, nw, wg, wu):
    RB, CB = 8, 2
    rb, cb = L // RB, S // CB
    return pl.pallas_call(
        _shared1_kernel,
        grid=(CB, RB),
        in_specs=[
            pl.BlockSpec((rb, D), lambda c, r: (r, 0)),
            pl.BlockSpec((1, D), lambda c, r: (0, 0)),
            pl.BlockSpec((cb, D), lambda c, r: (c, 0)),
            pl.BlockSpec((cb, D), lambda c, r: (c, 0)),
        ],
        out_specs=pl.BlockSpec((rb, cb), lambda c, r: (r, c)),
        out_shape=jax.ShapeDtypeStruct((L, S), jnp.float32),
    )(x, nw.reshape(1, D), wg.astype(jnp.bfloat16), wu.astype(jnp.bfloat16))


def _final_kernel(s1_ref, wd_ref, ident_ref, y_ref, gl_ref, o_ref):
    shared = _dot(s1_ref[...].astype(jnp.bfloat16), wd_ref[...].T)
    ident = ident_ref[...]
    gate = _sigmoid(gl_ref[:, 0:1])
    o_ref[...] = ident + gate * (y_ref[...] + shared)


def _final(s1, wd, ident, y, gl):
    RB = 8
    rb = L // RB
    return pl.pallas_call(
        _final_kernel,
        grid=(RB,),
        in_specs=[
            pl.BlockSpec((rb, S), lambda r: (r, 0)),
            pl.BlockSpec((D, S), lambda r: (0, 0)),
            pl.BlockSpec((rb, D), lambda r: (r, 0)),
            pl.BlockSpec((rb, D), lambda r: (r, 0)),
            pl.BlockSpec((rb, 128), lambda r: (r, 0)),
        ],
        out_specs=pl.BlockSpec((rb, D), lambda r: (r, 0)),
        out_shape=jax.ShapeDtypeStruct((L, D), jnp.float32),
    )(s1, wd.astype(jnp.bfloat16), ident, y, gl)


def kernel(hidden_states, context_norm_w, attn_in_w, attn_in_b, attn_out_w,
           attn_out_b, gate_norm_w, gate_w, expert_norm_w, expert_wg,
           expert_wu, expert_wd, shared_norm_w, shared_wg, shared_wu,
           shared_wd, out_gate_w, out_gate_b):
    x = hidden_states.reshape(L, D)
    qkv = _qkv(x, context_norm_w, attn_in_w, attn_in_b)
    ctx = _attn(qkv)
    ident = _outproj(ctx, attn_out_w, attn_out_b, x)
    base, coef, gl = _gate(ident, gate_norm_w, gate_w, out_gate_w, out_gate_b)
    y = _moe_dense(base, coef, expert_norm_w, expert_wg, expert_wu, expert_wd)
    s1 = _shared1(ident, shared_norm_w, shared_wg, shared_wu)
    out = _final(s1, shared_wd, ident, y, gl)
    return out.reshape(B, L, D)


# speedup vs baseline: 1.7850x; 1.0006x over previous
import functools

import jax
import jax.numpy as jnp
import numpy as np
from jax.experimental import pallas as pl
from jax.experimental.pallas import tpu as pltpu

B, L, D = 1, 2048, 2048
E, K, F = 8, 2, 1024
S = 2 * F
H = 4
HD = D // H


def _dot(a, b):
    return jnp.dot(a.astype(jnp.bfloat16), b.astype(jnp.bfloat16),
                   preferred_element_type=jnp.float32)


def _sigmoid(x):
    return 1.0 / (1.0 + jnp.exp(-x))


def _qkv_kernel(x_ref, nw_ref, w_ref, b_ref, o_ref):
    x = x_ref[...]
    v = jnp.mean(x * x, axis=-1, keepdims=True)
    xn = nw_ref[...] * (x * jax.lax.rsqrt(v + 1e-6))
    o_ref[...] = _dot(xn, w_ref[...].T) + b_ref[...]


def _qkv(x, nw, w, bias):
    RB, CB = 8, 6
    rb, cb = L // RB, (3 * D) // CB
    return pl.pallas_call(
        _qkv_kernel,
        grid=(CB, RB),
        in_specs=[
            pl.BlockSpec((rb, D), lambda c, r: (r, 0)),
            pl.BlockSpec((1, D), lambda c, r: (0, 0)),
            pl.BlockSpec((cb, D), lambda c, r: (c, 0)),
            pl.BlockSpec((1, cb), lambda c, r: (0, c)),
        ],
        out_specs=pl.BlockSpec((rb, cb), lambda c, r: (r, c)),
        out_shape=jax.ShapeDtypeStruct((L, 3 * D), jnp.float32),
    )(x, nw.reshape(1, D), w, bias.reshape(1, 3 * D))


def _attn_kernel(q_ref, k_ref, v_ref, o_ref):
    q = q_ref[...]
    k = k_ref[...]
    s = _dot(q, k.T) * (1.0 / np.sqrt(HD))
    s = s - jnp.max(s, axis=-1, keepdims=True)
    p = jnp.exp(s)
    den = jnp.sum(p, axis=-1, keepdims=True)
    o_ref[...] = _dot(p, v_ref[...]) / den


def _attn(qkv):
    QB = 2
    qb = L // QB
    return pl.pallas_call(
        _attn_kernel,
        grid=(H, QB),
        in_specs=[
            pl.BlockSpec((qb, HD), lambda h, q: (q, h)),
            pl.BlockSpec((L, HD), lambda h, q: (0, H + h)),
            pl.BlockSpec((L, HD), lambda h, q: (0, 2 * H + h)),
        ],
        out_specs=pl.BlockSpec((qb, HD), lambda h, q: (q, h)),
        out_shape=jax.ShapeDtypeStruct((L, D), jnp.float32),
    )(qkv, qkv, qkv)


def _outproj_kernel(ctx_ref, w_ref, b_ref, x_ref, o_ref):
    o_ref[...] = x_ref[...] + _dot(ctx_ref[...], w_ref[...].T) + b_ref[...]


def _outproj(ctx, w, bias, x):
    RB = 8
    rb = L // RB
    return pl.pallas_call(
        _outproj_kernel,
        grid=(RB,),
        in_specs=[
            pl.BlockSpec((rb, D), lambda r: (r, 0)),
            pl.BlockSpec((D, D), lambda r: (0, 0)),
            pl.BlockSpec((1, D), lambda r: (0, 0)),
            pl.BlockSpec((rb, D), lambda r: (r, 0)),
        ],
        out_specs=pl.BlockSpec((rb, D), lambda r: (r, 0)),
        out_shape=jax.ShapeDtypeStruct((L, D), jnp.float32),
    )(ctx, w, bias.reshape(1, D), x)


def _gate_kernel(hs_ref, gnw_ref, gw_ref, ogw_ref, ogb_ref,
                 base_ref, coef_ref, gl_ref):
    hs = hs_ref[...]
    v = jnp.mean(hs * hs, axis=-1, keepdims=True)
    base = hs * jax.lax.rsqrt(v + 1e-6)
    base_ref[...] = base
    logits = _dot(base * gnw_ref[...], gw_ref[...].T)
    m = jnp.max(logits, axis=-1, keepdims=True)
    p = jnp.exp(logits - m)
    scores = p / jnp.sum(p, axis=-1, keepdims=True)
    iota = jax.lax.broadcasted_iota(jnp.int32, (L, E), 1)
    m1 = jnp.max(scores, axis=-1, keepdims=True)
    i1 = jnp.min(jnp.where(scores == m1, iota, E), axis=-1, keepdims=True)
    s2 = jnp.where(iota == i1, -jnp.inf, scores)
    m2 = jnp.max(s2, axis=-1, keepdims=True)
    i2 = jnp.min(jnp.where(s2 == m2, iota, E), axis=-1, keepdims=True)
    wsum = m1 + m2 + 1e-20
    coef_ref[...] = (jnp.where(iota == i1, m1, 0.0)
                     + jnp.where(iota == i2, m2, 0.0)) / wsum
    gl = jnp.sum(hs * ogw_ref[...], axis=-1, keepdims=True) + ogb_ref[0, 0]
    gl_ref[...] = jnp.broadcast_to(gl, (L, 128))


def _gate(hs, gnw, gw, ogw, ogb):
    return pl.pallas_call(
        _gate_kernel,
        grid=(1,),
        in_specs=[
            pl.BlockSpec((L, D), lambda i: (0, 0)),
            pl.BlockSpec((1, D), lambda i: (0, 0)),
            pl.BlockSpec((E, D), lambda i: (0, 0)),
            pl.BlockSpec((1, D), lambda i: (0, 0)),
            pl.BlockSpec((1, 1), lambda i: (0, 0)),
        ],
        out_specs=[
            pl.BlockSpec((L, D), lambda i: (0, 0)),
            pl.BlockSpec((L, E), lambda i: (0, 0)),
            pl.BlockSpec((L, 128), lambda i: (0, 0)),
        ],
        out_shape=[
            jax.ShapeDtypeStruct((L, D), jnp.float32),
            jax.ShapeDtypeStruct((L, E), jnp.float32),
            jax.ShapeDtypeStruct((L, 128), jnp.float32),
        ],
    )(hs, gnw.reshape(1, D), gw, ogw.reshape(1, D), ogb.reshape(1, 1))


def _moe_dense_kernel(base_ref, coef_ref, enw_ref, wg_ref, wu_ref, wd_ref, y_ref):
    e = pl.program_id(1)
    xn = (base_ref[...] * enw_ref[0]).astype(jnp.bfloat16)
    g = _dot(xn, wg_ref[0].T)
    g = g * _sigmoid(g)
    u = _dot(xn, wu_ref[0].T)
    eo = _dot((g * u).astype(jnp.bfloat16), wd_ref[0].T)
    iota = jax.lax.broadcasted_iota(jnp.int32, (1, E), 1)
    ce = jnp.sum(coef_ref[...] * (iota == e), axis=-1, keepdims=True)
    contrib = ce * eo

    @pl.when(e == 0)
    def _():
        y_ref[...] = contrib

    @pl.when(e > 0)
    def _():
        y_ref[...] += contrib


def _moe_dense(base, coef, enw, wg, wu, wd):
    RB = 4
    rb = L // RB
    return pl.pallas_call(
        _moe_dense_kernel,
        grid=(RB, E),
        in_specs=[
            pl.BlockSpec((rb, D), lambda r, e: (r, 0)),
            pl.BlockSpec((rb, E), lambda r, e: (r, 0)),
            pl.BlockSpec((1, D), lambda r, e: (0, e)),
            pl.BlockSpec((1, F, D), lambda r, e: (e, 0, 0)),
            pl.BlockSpec((1, F, D), lambda r, e: (e, 0, 0)),
            pl.BlockSpec((1, D, F), lambda r, e: (e, 0, 0)),
        ],
        out_specs=pl.BlockSpec((rb, D), lambda r, e: (r, 0)),
        out_shape=jax.ShapeDtypeStruct((L, D), jnp.float32),
    )(base, coef, enw.reshape(1, E * D),
      wg.astype(jnp.bfloat16), wu.astype(jnp.bfloat16), wd.astype(jnp.bfloat16))


def _shared1_kernel(x_ref, nw_ref, wg_ref, wu_ref, o_ref):
    x = x_ref[...]
    v = jnp.mean(x * x, axis=-1, keepdims=True)
    xn = (nw_ref[...] * (x * jax.lax.rsqrt(v + 1e-6))).astype(jnp.bfloat16)
    g = _dot(xn, wg_ref[...].T)
    o_ref[...] = g * _sigmoid(g) * _dot(xn, wu_ref[...].T)


def _shared1(x, nw, wg, wu):
    RB, CB = 8, 2
    rb, cb = L // RB, S // CB
    return pl.pallas_call(
        _shared1_kernel,
        grid=(CB, RB),
        in_specs=[
            pl.BlockSpec((rb, D), lambda c, r: (r, 0)),
            pl.BlockSpec((1, D), lambda c, r: (0, 0)),
            pl.BlockSpec((cb, D), lambda c, r: (c, 0)),
            pl.BlockSpec((cb, D), lambda c, r: (c, 0)),
        ],
        out_specs=pl.BlockSpec((rb, cb), lambda c, r: (r, c)),
        out_shape=jax.ShapeDtypeStruct((L, S), jnp.float32),
    )(x, nw.reshape(1, D), wg.astype(jnp.bfloat16), wu.astype(jnp.bfloat16))


def _final_kernel(s1_ref, wd_ref, ident_ref, y_ref, gl_ref, o_ref):
    shared = _dot(s1_ref[...].astype(jnp.bfloat16), wd_ref[...].T)
    ident = ident_ref[...]
    gate = _sigmoid(gl_ref[:, 0:1])
    o_ref[...] = ident + gate * (y_ref[...] + shared)


def _final(s1, wd, ident, y, gl):
    RB = 8
    rb = L // RB
    return pl.pallas_call(
        _final_kernel,
        grid=(RB,),
        in_specs=[
            pl.BlockSpec((rb, S), lambda r: (r, 0)),
            pl.BlockSpec((D, S), lambda r: (0, 0)),
            pl.BlockSpec((rb, D), lambda r: (r, 0)),
            pl.BlockSpec((rb, D), lambda r: (r, 0)),
            pl.BlockSpec((rb, 128), lambda r: (r, 0)),
        ],
        out_specs=pl.BlockSpec((rb, D), lambda r: (r, 0)),
        out_shape=jax.ShapeDtypeStruct((L, D), jnp.float32),
    )(s1, wd.astype(jnp.bfloat16), ident, y, gl)


def kernel(hidden_states, context_norm_w, attn_in_w, attn_in_b, attn_out_w,
           attn_out_b, gate_norm_w, gate_w, expert_norm_w, expert_wg,
           expert_wu, expert_wd, shared_norm_w, shared_wg, shared_wu,
           shared_wd, out_gate_w, out_gate_b):
    x = hidden_states.reshape(L, D)
    qkv = _qkv(x, context_norm_w, attn_in_w, attn_in_b)
    ctx = _attn(qkv)
    ident = _outproj(ctx, attn_out_w, attn_out_b, x)
    base, coef, gl = _gate(ident, gate_norm_w, gate_w, out_gate_w, out_gate_b)
    y = _moe_dense(base, coef, expert_norm_w, expert_wg, expert_wu, expert_wd)
    s1 = _shared1(ident, shared_norm_w, shared_wg, shared_wu)
    out = _final(s1, shared_wd, ident, y, gl)
    return out.reshape(B, L, D)
